# baseline (speedup 1.0000x reference)
import jax
import jax.numpy as jnp
from jax import lax
from jax.experimental import pallas as pl
from jax.experimental.pallas import tpu as pltpu

H_HEADS = 4
HP = 8
C_IN = 192
C_OUT = 192
E_EDGES = 200
EP = 256
BN = 3584
N_TOT = 224 * 224
NB = N_TOT // BN


def _prep_body(w1r_ref, b1r_ref, att_ref, w2r_ref, vt_ref, gt_ref, cb_ref, m_ref):
    zeros4 = jnp.zeros((H_HEADS, C_IN), dtype=jnp.float32)
    vrows = [jnp.dot(att_ref[h : h + 1, :], w1r_ref[h],
                     preferred_element_type=jnp.float32)
             for h in range(H_HEADS)]
    vt_ref[...] = lax.concatenate(vrows + [zeros4], 0)
    grows = [lax.dot_general(b1r_ref[h : h + 1, :], w2r_ref[h],
                             (((1,), (1,)), ((), ())),
                             preferred_element_type=jnp.float32)
             for h in range(H_HEADS)]
    gt_ref[...] = lax.concatenate(grows + [zeros4], 0)
    cbv = jnp.sum(att_ref[...] * b1r_ref[...], axis=1, keepdims=True)
    cb8 = lax.concatenate([cbv, jnp.zeros((H_HEADS, 1), jnp.float32)], 0)
    cb_ref[...] = jnp.broadcast_to(cb8, (HP, 128))
    for h in range(H_HEADS):
        m_ref[h] = jnp.dot(w2r_ref[h], w1r_ref[h],
                           preferred_element_type=jnp.float32)


def _fused_body(x_ref, idxr_ref, idxc_ref, vt_ref, cb_ref, m_ref, gt_ref,
                b2_ref, wr_ref, br_ref, o_ref, xbf_ref, ys_ref, ss_ref,
                et_ref):
    p = pl.program_id(1)
    j = pl.program_id(2)

    @pl.when(p == 0)
    def _scatter():
        @pl.when(j == 0)
        def _init():
            ys_ref[...] = jnp.zeros_like(ys_ref)
            ss_ref[...] = jnp.zeros_like(ss_ref)

        idxc = idxc_ref[0, 0]
        oh = (lax.broadcasted_iota(jnp.int32, (BN, EP), 1) == idxc
              ).astype(jnp.bfloat16)
        xb = x_ref[0].astype(jnp.bfloat16)
        xbf_ref[j] = xb
        vtb = vt_ref[...].astype(jnp.bfloat16)
        logits = jnp.dot(vtb, xb, preferred_element_type=jnp.float32)
        logits = logits + cb_ref[:, :1]
        ex = jnp.exp(logits)
        exb = ex.astype(jnp.bfloat16)
        ss_ref[...] += jnp.dot(exb, oh, preferred_element_type=jnp.float32)
        for h in range(H_HEADS):
            ys_ref[h] += jnp.dot(xb * exb[h : h + 1, :], oh,
                                 preferred_element_type=jnp.float32)

    @pl.when((p == 1) & (j == 0))
    def _edge():
        s = ss_ref[...]
        cnt = jnp.maximum(s[H_HEADS : H_HEADS + 1, :], 1.0)
        cinv = 1.0 / cnt
        dn = jnp.sum(s, axis=1, keepdims=True)
        sn = s * (cinv / dn)
        acc = lax.dot_general(gt_ref[...], sn, (((0,), (0,)), ((), ())),
                              preferred_element_type=jnp.float32)
        for h in range(H_HEADS):
            dnh = jnp.sum(s[h : h + 1, :], axis=1, keepdims=True)
            zh = ys_ref[h] * (cinv / dnh)
            acc = acc + jnp.dot(m_ref[h], zh,
                                preferred_element_type=jnp.float32)
        et_ref[...] = acc + b2_ref[...]

    @pl.when(p == 1)
    def _output():
        idxv = idxr_ref[0, 0]
        oht = (lax.broadcasted_iota(jnp.int32, (EP, BN), 0) == idxv
               ).astype(jnp.bfloat16)
        etb = et_ref[...].astype(jnp.bfloat16)
        g = jnp.dot(etb, oht, preferred_element_type=jnp.float32)
        wrb = wr_ref[...].astype(jnp.bfloat16)
        r = jnp.dot(wrb, xbf_ref[j], preferred_element_type=jnp.float32)
        v = g + r + br_ref[...]
        o_ref[0] = jnp.where(v > 0, v, jnp.exp(jnp.minimum(v, 0.0)) - 1.0)


def kernel(x, hyperedge_labels, W1, b1, W2, b2, attention, Wr, br):
    B, C, H, W = x.shape
    N = H * W
    x3 = x.reshape(B, C, N)
    idx32 = hyperedge_labels.astype(jnp.int32)
    idxr = idx32.reshape(B, NB, 1, BN)
    idxc = idx32.reshape(B, NB, BN, 1)

    w1r = W1.reshape(H_HEADS, C_OUT, C)
    b1r = b1.reshape(H_HEADS, C_OUT)
    att = attention.reshape(H_HEADS, C_OUT)
    w2r = jnp.transpose(W2.reshape(C_OUT, H_HEADS, C_OUT), (1, 0, 2))
    b2c = b2.reshape(C_OUT, 1)
    brc = br.reshape(C_OUT, 1)

    vt, gt, cb, m = pl.pallas_call(
        _prep_body,
        out_shape=[
            jax.ShapeDtypeStruct((HP, C_IN), jnp.float32),
            jax.ShapeDtypeStruct((HP, C_IN), jnp.float32),
            jax.ShapeDtypeStruct((HP, 128), jnp.float32),
            jax.ShapeDtypeStruct((H_HEADS, C_OUT, C_IN), jnp.float32),
        ],
    )(w1r, b1r, att, w2r)

    out = pl.pallas_call(
        _fused_body,
        grid=(B, 2, NB),
        in_specs=[
            pl.BlockSpec((1, C, BN),
                         lambda b, p, j: (b, 0, j * (1 - p) + (NB - 1) * p)),
            pl.BlockSpec((1, 1, 1, BN), lambda b, p, j: (b, j, 0, 0)),
            pl.BlockSpec((1, 1, BN, 1), lambda b, p, j: (b, j, 0, 0)),
            pl.BlockSpec((HP, C_IN), lambda b, p, j: (0, 0)),
            pl.BlockSpec((HP, 128), lambda b, p, j: (0, 0)),
            pl.BlockSpec((H_HEADS, C_OUT, C_IN), lambda b, p, j: (0, 0, 0)),
            pl.BlockSpec((HP, C_IN), lambda b, p, j: (0, 0)),
            pl.BlockSpec((C_OUT, 1), lambda b, p, j: (0, 0)),
            pl.BlockSpec((C_OUT, C_IN), lambda b, p, j: (0, 0)),
            pl.BlockSpec((C_OUT, 1), lambda b, p, j: (0, 0)),
        ],
        out_specs=pl.BlockSpec((1, C_OUT, BN), lambda b, p, j: (b, 0, j)),
        out_shape=jax.ShapeDtypeStruct((B, C_OUT, N), jnp.float32),
        scratch_shapes=[
            pltpu.VMEM((NB, C_IN, BN), jnp.bfloat16),
            pltpu.VMEM((H_HEADS, C_IN, EP), jnp.float32),
            pltpu.VMEM((HP, EP), jnp.float32),
            pltpu.VMEM((C_OUT, EP), jnp.float32),
        ],
        compiler_params=pltpu.CompilerParams(
            vmem_limit_bytes=128 * 1024 * 1024,
        ),
    )(x3, idxr, idxc, vt, cb, m, gt, b2c, Wr, brc)

    return out.reshape(B, C_OUT, H, W)


# speedup vs baseline: 1.1413x; 1.1413x over previous
import jax
import jax.numpy as jnp
from jax import lax
from jax.experimental import pallas as pl
from jax.experimental.pallas import tpu as pltpu

H_HEADS = 4
HP = 8
C_IN = 192
C_OUT = 192
E_EDGES = 200
EP = 256
BN = 3584
N_TOT = 224 * 224
NB = N_TOT // BN


def _prep_body(w1r_ref, b1r_ref, att_ref, w2r_ref, vt_ref, gt_ref, cb_ref, m_ref):
    zeros4 = jnp.zeros((H_HEADS, C_IN), dtype=jnp.float32)
    vrows = [jnp.dot(att_ref[h : h + 1, :], w1r_ref[h],
                     preferred_element_type=jnp.float32)
             for h in range(H_HEADS)]
    vt_ref[...] = lax.concatenate(vrows + [zeros4], 0)
    grows = [lax.dot_general(b1r_ref[h : h + 1, :], w2r_ref[h],
                             (((1,), (1,)), ((), ())),
                             preferred_element_type=jnp.float32)
             for h in range(H_HEADS)]
    gt_ref[...] = lax.concatenate(grows + [zeros4], 0)
    cbv = jnp.sum(att_ref[...] * b1r_ref[...], axis=1, keepdims=True)
    cb8 = lax.concatenate([cbv, jnp.zeros((H_HEADS, 1), jnp.float32)], 0)
    cb_ref[...] = jnp.broadcast_to(cb8, (HP, 128))
    for h in range(H_HEADS):
        m_ref[h] = jnp.dot(w2r_ref[h], w1r_ref[h],
                           preferred_element_type=jnp.float32)


def _fused_body(x_ref, idxr_ref, vt_ref, cb_ref, m_ref, gt_ref,
                b2_ref, wr_ref, br_ref, o_ref, xbf_ref, ys_ref, ss_ref,
                et_ref):
    p = pl.program_id(1)
    j = pl.program_id(2)

    @pl.when(p == 0)
    def _scatter():
        @pl.when(j == 0)
        def _init():
            ys_ref[...] = jnp.zeros_like(ys_ref)
            ss_ref[...] = jnp.zeros_like(ss_ref)

        idxv = idxr_ref[0, 0]
        oht = (lax.broadcasted_iota(jnp.int32, (EP, BN), 0) == idxv
               ).astype(jnp.bfloat16)
        xb = x_ref[0].astype(jnp.bfloat16)
        xbf_ref[j] = xb
        vtb = vt_ref[...].astype(jnp.bfloat16)
        logits = jnp.dot(vtb, xb, preferred_element_type=jnp.float32)
        logits = logits + cb_ref[:, :1]
        ex = jnp.exp(logits)
        exb = ex.astype(jnp.bfloat16)
        ss_ref[...] += lax.dot_general(exb, oht, (((1,), (1,)), ((), ())),
                                       preferred_element_type=jnp.float32)
        for h in range(H_HEADS):
            ys_ref[h] += lax.dot_general(xb * exb[h : h + 1, :], oht,
                                         (((1,), (1,)), ((), ())),
                                         preferred_element_type=jnp.float32)

    @pl.when((p == 1) & (j == 0))
    def _edge():
        s = ss_ref[...]
        cnt = jnp.maximum(s[H_HEADS : H_HEADS + 1, :], 1.0)
        cinv = 1.0 / cnt
        dn = jnp.sum(s, axis=1, keepdims=True)
        sn = s * (cinv / dn)
        acc = lax.dot_general(gt_ref[...], sn, (((0,), (0,)), ((), ())),
                              preferred_element_type=jnp.float32)
        for h in range(H_HEADS):
            dnh = jnp.sum(s[h : h + 1, :], axis=1, keepdims=True)
            zh = ys_ref[h] * (cinv / dnh)
            acc = acc + jnp.dot(m_ref[h], zh,
                                preferred_element_type=jnp.float32)
        et_ref[...] = acc + b2_ref[...]

    @pl.when(p == 1)
    def _output():
        idxv = idxr_ref[0, 0]
        oht = (lax.broadcasted_iota(jnp.int32, (EP, BN), 0) == idxv
               ).astype(jnp.bfloat16)
        etb = et_ref[...].astype(jnp.bfloat16)
        g = jnp.dot(etb, oht, preferred_element_type=jnp.float32)
        wrb = wr_ref[...].astype(jnp.bfloat16)
        r = jnp.dot(wrb, xbf_ref[j], preferred_element_type=jnp.float32)
        v = g + r + br_ref[...]
        o_ref[0] = jnp.where(v > 0, v, jnp.exp(jnp.minimum(v, 0.0)) - 1.0)


def kernel(x, hyperedge_labels, W1, b1, W2, b2, attention, Wr, br):
    B, C, H, W = x.shape
    N = H * W
    x3 = x.reshape(B, C, N)
    idx32 = hyperedge_labels.astype(jnp.int32)
    idxr = idx32.reshape(B, NB, 1, BN)
    idxc = idx32.reshape(B, NB, BN, 1)

    w1r = W1.reshape(H_HEADS, C_OUT, C)
    b1r = b1.reshape(H_HEADS, C_OUT)
    att = attention.reshape(H_HEADS, C_OUT)
    w2r = jnp.transpose(W2.reshape(C_OUT, H_HEADS, C_OUT), (1, 0, 2))
    b2c = b2.reshape(C_OUT, 1)
    brc = br.reshape(C_OUT, 1)

    vt, gt, cb, m = pl.pallas_call(
        _prep_body,
        out_shape=[
            jax.ShapeDtypeStruct((HP, C_IN), jnp.float32),
            jax.ShapeDtypeStruct((HP, C_IN), jnp.float32),
            jax.ShapeDtypeStruct((HP, 128), jnp.float32),
            jax.ShapeDtypeStruct((H_HEADS, C_OUT, C_IN), jnp.float32),
        ],
    )(w1r, b1r, att, w2r)

    out = pl.pallas_call(
        _fused_body,
        grid=(B, 2, NB),
        in_specs=[
            pl.BlockSpec((1, C, BN),
                         lambda b, p, j: (b, 0, j * (1 - p) + (NB - 1) * p)),
            pl.BlockSpec((1, 1, 1, BN), lambda b, p, j: (b, j, 0, 0)),
            pl.BlockSpec((HP, C_IN), lambda b, p, j: (0, 0)),
            pl.BlockSpec((HP, 128), lambda b, p, j: (0, 0)),
            pl.BlockSpec((H_HEADS, C_OUT, C_IN), lambda b, p, j: (0, 0, 0)),
            pl.BlockSpec((HP, C_IN), lambda b, p, j: (0, 0)),
            pl.BlockSpec((C_OUT, 1), lambda b, p, j: (0, 0)),
            pl.BlockSpec((C_OUT, C_IN), lambda b, p, j: (0, 0)),
            pl.BlockSpec((C_OUT, 1), lambda b, p, j: (0, 0)),
        ],
        out_specs=pl.BlockSpec((1, C_OUT, BN), lambda b, p, j: (b, 0, j)),
        out_shape=jax.ShapeDtypeStruct((B, C_OUT, N), jnp.float32),
        scratch_shapes=[
            pltpu.VMEM((NB, C_IN, BN), jnp.bfloat16),
            pltpu.VMEM((H_HEADS, C_IN, EP), jnp.float32),
            pltpu.VMEM((HP, EP), jnp.float32),
            pltpu.VMEM((C_OUT, EP), jnp.float32),
        ],
        compiler_params=pltpu.CompilerParams(
            vmem_limit_bytes=128 * 1024 * 1024,
        ),
    )(x3, idxr, vt, cb, m, gt, b2c, Wr, brc)

    return out.reshape(B, C_OUT, H, W)


# speedup vs baseline: 1.1418x; 1.0004x over previous
import jax
import jax.numpy as jnp
from jax import lax
from jax.experimental import pallas as pl
from jax.experimental.pallas import tpu as pltpu

H_HEADS = 4
HP = 8
C_IN = 192
C_OUT = 192
E_EDGES = 200
EP = 256
BN = 3584
N_TOT = 224 * 224
NB = N_TOT // BN


def _fused_body(x_ref, idxr_ref, w1r_ref, b1r_ref, att_ref, w2r_ref,
                b2_ref, wr_ref, br_ref, o_ref, xbf_ref, ys_ref, ss_ref,
                et_ref, vt_ref, cb_ref, m_ref, gt_ref):
    b = pl.program_id(0)
    p = pl.program_id(1)
    j = pl.program_id(2)

    @pl.when((b == 0) & (p == 0) & (j == 0))
    def _prep():
        zeros4 = jnp.zeros((H_HEADS, C_IN), dtype=jnp.float32)
        vrows = [jnp.dot(att_ref[h : h + 1, :], w1r_ref[h],
                         preferred_element_type=jnp.float32)
                 for h in range(H_HEADS)]
        vt_ref[...] = lax.concatenate(vrows + [zeros4], 0)
        grows = [lax.dot_general(b1r_ref[h : h + 1, :], w2r_ref[h],
                                 (((1,), (1,)), ((), ())),
                                 preferred_element_type=jnp.float32)
                 for h in range(H_HEADS)]
        gt_ref[...] = lax.concatenate(grows + [zeros4], 0)
        cbv = jnp.sum(att_ref[...] * b1r_ref[...], axis=1, keepdims=True)
        cb8 = lax.concatenate([cbv, jnp.zeros((H_HEADS, 1), jnp.float32)], 0)
        cb_ref[...] = jnp.broadcast_to(cb8, (HP, 128))
        for h in range(H_HEADS):
            m_ref[h] = jnp.dot(w2r_ref[h], w1r_ref[h],
                               preferred_element_type=jnp.float32)

    @pl.when(p == 0)
    def _scatter():
        @pl.when(j == 0)
        def _init():
            ys_ref[...] = jnp.zeros_like(ys_ref)
            ss_ref[...] = jnp.zeros_like(ss_ref)

        idxv = idxr_ref[0, 0]
        oht = (lax.broadcasted_iota(jnp.int32, (EP, BN), 0) == idxv
               ).astype(jnp.bfloat16)
        xb = x_ref[0].astype(jnp.bfloat16)
        xbf_ref[j] = xb
        vtb = vt_ref[...].astype(jnp.bfloat16)
        logits = jnp.dot(vtb, xb, preferred_element_type=jnp.float32)
        logits = logits + cb_ref[:, :1]
        ex = jnp.exp(logits)
        exb = ex.astype(jnp.bfloat16)
        ss_ref[...] += lax.dot_general(exb, oht, (((1,), (1,)), ((), ())),
                                       preferred_element_type=jnp.float32)
        for h in range(H_HEADS):
            ys_ref[h] += lax.dot_general(xb * exb[h : h + 1, :], oht,
                                         (((1,), (1,)), ((), ())),
                                         preferred_element_type=jnp.float32)

    @pl.when((p == 1) & (j == 0))
    def _edge():
        s = ss_ref[...]
        cnt = jnp.maximum(s[H_HEADS : H_HEADS + 1, :], 1.0)
        cinv = 1.0 / cnt
        dn = jnp.sum(s, axis=1, keepdims=True)
        sn = s * (cinv / dn)
        acc = lax.dot_general(gt_ref[...], sn, (((0,), (0,)), ((), ())),
                              preferred_element_type=jnp.float32)
        for h in range(H_HEADS):
            dnh = jnp.sum(s[h : h + 1, :], axis=1, keepdims=True)
            zh = ys_ref[h] * (cinv / dnh)
            acc = acc + jnp.dot(m_ref[h], zh,
                                preferred_element_type=jnp.float32)
        et_ref[...] = acc + b2_ref[...]

    @pl.when(p == 1)
    def _output():
        idxv = idxr_ref[0, 0]
        oht = (lax.broadcasted_iota(jnp.int32, (EP, BN), 0) == idxv
               ).astype(jnp.bfloat16)
        etb = et_ref[...].astype(jnp.bfloat16)
        g = jnp.dot(etb, oht, preferred_element_type=jnp.float32)
        wrb = wr_ref[...].astype(jnp.bfloat16)
        r = jnp.dot(wrb, xbf_ref[j], preferred_element_type=jnp.float32)
        v = g + r + br_ref[...]
        o_ref[0] = jnp.where(v > 0, v, jnp.exp(jnp.minimum(v, 0.0)) - 1.0)


def kernel(x, hyperedge_labels, W1, b1, W2, b2, attention, Wr, br):
    B, C, H, W = x.shape
    N = H * W
    x3 = x.reshape(B, C, N)
    idx32 = hyperedge_labels.astype(jnp.int32)
    idxr = idx32.reshape(B, NB, 1, BN)
    idxc = idx32.reshape(B, NB, BN, 1)

    w1r = W1.reshape(H_HEADS, C_OUT, C)
    b1r = b1.reshape(H_HEADS, C_OUT)
    att = attention.reshape(H_HEADS, C_OUT)
    w2r = jnp.transpose(W2.reshape(C_OUT, H_HEADS, C_OUT), (1, 0, 2))
    b2c = b2.reshape(C_OUT, 1)
    brc = br.reshape(C_OUT, 1)

    out = pl.pallas_call(
        _fused_body,
        grid=(B, 2, NB),
        in_specs=[
            pl.BlockSpec((1, C, BN),
                         lambda b, p, j: (b, 0, j * (1 - p) + (NB - 1) * p)),
            pl.BlockSpec((1, 1, 1, BN), lambda b, p, j: (b, j, 0, 0)),
            pl.BlockSpec((H_HEADS, C_OUT, C_IN), lambda b, p, j: (0, 0, 0)),
            pl.BlockSpec((H_HEADS, C_OUT), lambda b, p, j: (0, 0)),
            pl.BlockSpec((H_HEADS, C_OUT), lambda b, p, j: (0, 0)),
            pl.BlockSpec((H_HEADS, C_OUT, C_IN), lambda b, p, j: (0, 0, 0)),
            pl.BlockSpec((C_OUT, 1), lambda b, p, j: (0, 0)),
            pl.BlockSpec((C_OUT, C_IN), lambda b, p, j: (0, 0)),
            pl.BlockSpec((C_OUT, 1), lambda b, p, j: (0, 0)),
        ],
        out_specs=pl.BlockSpec((1, C_OUT, BN), lambda b, p, j: (b, 0, j)),
        out_shape=jax.ShapeDtypeStruct((B, C_OUT, N), jnp.float32),
        scratch_shapes=[
            pltpu.VMEM((NB, C_IN, BN), jnp.bfloat16),
            pltpu.VMEM((H_HEADS, C_IN, EP), jnp.float32),
            pltpu.VMEM((HP, EP), jnp.float32),
            pltpu.VMEM((C_OUT, EP), jnp.float32),
            pltpu.VMEM((HP, C_IN), jnp.float32),
            pltpu.VMEM((HP, 128), jnp.float32),
            pltpu.VMEM((H_HEADS, C_OUT, C_IN), jnp.float32),
            pltpu.VMEM((HP, C_IN), jnp.float32),
        ],
        compiler_params=pltpu.CompilerParams(
            vmem_limit_bytes=128 * 1024 * 1024,
        ),
    )(x3, idxr, w1r, b1r, att, w2r, b2c, Wr, brc)

    return out.reshape(B, C_OUT, H, W)


# speedup vs baseline: 1.2645x; 1.1075x over previous
import jax
import jax.numpy as jnp
from jax import lax
from jax.experimental import pallas as pl
from jax.experimental.pallas import tpu as pltpu

H_HEADS = 4
HP = 8
C_IN = 192
C_OUT = 192
E_EDGES = 200
EP = 256
BN = 3584
N_TOT = 224 * 224
NB = N_TOT // BN


def _fused_body(x_ref, idxr_ref, w1r_ref, b1r_ref, att_ref, w2r_ref,
                b2_ref, wr_ref, br_ref, o_ref, xbf_ref, ohts_ref, ys_ref,
                et_ref, vt_ref, cb_ref, m_ref, gt_ref):
    b = pl.program_id(0)
    p = pl.program_id(1)
    j = pl.program_id(2)

    @pl.when((b == 0) & (p == 0) & (j == 0))
    def _prep():
        zeros4 = jnp.zeros((H_HEADS, C_IN), dtype=jnp.float32)
        vrows = [jnp.dot(att_ref[h : h + 1, :], w1r_ref[h],
                         preferred_element_type=jnp.float32)
                 for h in range(H_HEADS)]
        vt_ref[...] = lax.concatenate(vrows + [zeros4], 0)
        grows = [lax.dot_general(b1r_ref[h : h + 1, :], w2r_ref[h],
                                 (((1,), (1,)), ((), ())),
                                 preferred_element_type=jnp.float32)
                 for h in range(H_HEADS)]
        gt_ref[...] = lax.concatenate(grows + [zeros4], 0)
        cbv = jnp.sum(att_ref[...] * b1r_ref[...], axis=1, keepdims=True)
        cb8 = lax.concatenate([cbv, jnp.zeros((H_HEADS, 1), jnp.float32)], 0)
        cb_ref[...] = jnp.broadcast_to(cb8, (HP, 128))
        for h in range(H_HEADS):
            m_ref[h] = jnp.dot(w2r_ref[h], w1r_ref[h],
                               preferred_element_type=jnp.float32)

    @pl.when(p == 0)
    def _scatter():
        @pl.when(j == 0)
        def _init():
            ys_ref[...] = jnp.zeros_like(ys_ref)

        idxv = idxr_ref[0, 0]
        oht = (lax.broadcasted_iota(jnp.int32, (EP, BN), 0) == idxv
               ).astype(jnp.bfloat16)
        ohts_ref[j] = oht
        xb = x_ref[0].astype(jnp.bfloat16)
        xbf_ref[j] = xb
        vtb = vt_ref[...].astype(jnp.bfloat16)
        logits = jnp.dot(vtb, xb, preferred_element_type=jnp.float32)
        logits = logits + cb_ref[:, :1]
        ex = jnp.exp(logits)
        exb = ex.astype(jnp.bfloat16)
        ls = lax.concatenate(
            [xb * exb[h : h + 1, :] for h in range(H_HEADS)] + [exb], 0)
        ys_ref[...] += lax.dot_general(ls, oht, (((1,), (1,)), ((), ())),
                                       preferred_element_type=jnp.float32)

    @pl.when((p == 1) & (j == 0))
    def _edge():
        s = ys_ref[4 * C_IN : 4 * C_IN + HP]
        cnt = jnp.maximum(s[H_HEADS : H_HEADS + 1, :], 1.0)
        cinv = 1.0 / cnt
        dn = jnp.sum(s, axis=1, keepdims=True)
        sn = s * (cinv / dn)
        acc = lax.dot_general(gt_ref[...], sn, (((0,), (0,)), ((), ())),
                              preferred_element_type=jnp.float32)
        for h in range(H_HEADS):
            dnh = jnp.sum(s[h : h + 1, :], axis=1, keepdims=True)
            zh = ys_ref[h * C_IN : (h + 1) * C_IN] * (cinv / dnh)
            acc = acc + jnp.dot(m_ref[h], zh,
                                preferred_element_type=jnp.float32)
        et_ref[...] = acc + b2_ref[...]

    @pl.when(p == 1)
    def _output():
        oht = ohts_ref[j]
        etb = et_ref[...].astype(jnp.bfloat16)
        g = jnp.dot(etb, oht, preferred_element_type=jnp.float32)
        wrb = wr_ref[...].astype(jnp.bfloat16)
        r = jnp.dot(wrb, xbf_ref[j], preferred_element_type=jnp.float32)
        v = g + r + br_ref[...]
        o_ref[0] = jnp.where(v > 0, v, jnp.exp(jnp.minimum(v, 0.0)) - 1.0)


def kernel(x, hyperedge_labels, W1, b1, W2, b2, attention, Wr, br):
    B, C, H, W = x.shape
    N = H * W
    x3 = x.reshape(B, C, N)
    idx32 = hyperedge_labels.astype(jnp.int32)
    idxr = idx32.reshape(B, NB, 1, BN)
    idxc = idx32.reshape(B, NB, BN, 1)

    w1r = W1.reshape(H_HEADS, C_OUT, C)
    b1r = b1.reshape(H_HEADS, C_OUT)
    att = attention.reshape(H_HEADS, C_OUT)
    w2r = jnp.transpose(W2.reshape(C_OUT, H_HEADS, C_OUT), (1, 0, 2))
    b2c = b2.reshape(C_OUT, 1)
    brc = br.reshape(C_OUT, 1)

    out = pl.pallas_call(
        _fused_body,
        grid=(B, 2, NB),
        in_specs=[
            pl.BlockSpec((1, C, BN),
                         lambda b, p, j: (b, 0, j * (1 - p) + (NB - 1) * p)),
            pl.BlockSpec((1, 1, 1, BN), lambda b, p, j: (b, j, 0, 0)),
            pl.BlockSpec((H_HEADS, C_OUT, C_IN), lambda b, p, j: (0, 0, 0)),
            pl.BlockSpec((H_HEADS, C_OUT), lambda b, p, j: (0, 0)),
            pl.BlockSpec((H_HEADS, C_OUT), lambda b, p, j: (0, 0)),
            pl.BlockSpec((H_HEADS, C_OUT, C_IN), lambda b, p, j: (0, 0, 0)),
            pl.BlockSpec((C_OUT, 1), lambda b, p, j: (0, 0)),
            pl.BlockSpec((C_OUT, C_IN), lambda b, p, j: (0, 0)),
            pl.BlockSpec((C_OUT, 1), lambda b, p, j: (0, 0)),
        ],
        out_specs=pl.BlockSpec((1, C_OUT, BN), lambda b, p, j: (b, 0, j)),
        out_shape=jax.ShapeDtypeStruct((B, C_OUT, N), jnp.float32),
        scratch_shapes=[
            pltpu.VMEM((NB, C_IN, BN), jnp.bfloat16),
            pltpu.VMEM((NB, EP, BN), jnp.bfloat16),
            pltpu.VMEM((H_HEADS * C_IN + HP, EP), jnp.float32),
            pltpu.VMEM((C_OUT, EP), jnp.float32),
            pltpu.VMEM((HP, C_IN), jnp.float32),
            pltpu.VMEM((HP, 128), jnp.float32),
            pltpu.VMEM((H_HEADS, C_OUT, C_IN), jnp.float32),
            pltpu.VMEM((HP, C_IN), jnp.float32),
        ],
        compiler_params=pltpu.CompilerParams(
            vmem_limit_bytes=128 * 1024 * 1024,
        ),
    )(x3, idxr, w1r, b1r, att, w2r, b2c, Wr, brc)

    return out.reshape(B, C_OUT, H, W)


# speedup vs baseline: 1.2760x; 1.0091x over previous
import jax
import jax.numpy as jnp
from jax import lax
from jax.experimental import pallas as pl
from jax.experimental.pallas import tpu as pltpu

H_HEADS = 4
HP = 8
C_IN = 192
C_OUT = 192
E_EDGES = 200
EP = 208
BN = 3584
N_TOT = 224 * 224
NB = N_TOT // BN


def _fused_body(x_ref, idxr_ref, w1r_ref, b1r_ref, att_ref, w2r_ref,
                b2_ref, wr_ref, br_ref, o_ref, xbf_ref, ohts_ref, ys_ref,
                et_ref, vt_ref, cb_ref, m_ref, gt_ref):
    b = pl.program_id(0)
    p = pl.program_id(1)
    j = pl.program_id(2)

    @pl.when((b == 0) & (p == 0) & (j == 0))
    def _prep():
        zeros4 = jnp.zeros((H_HEADS, C_IN), dtype=jnp.float32)
        vrows = [jnp.dot(att_ref[h : h + 1, :], w1r_ref[h],
                         preferred_element_type=jnp.float32)
                 for h in range(H_HEADS)]
        vt_ref[...] = lax.concatenate(vrows + [zeros4], 0)
        grows = [lax.dot_general(b1r_ref[h : h + 1, :], w2r_ref[h],
                                 (((1,), (1,)), ((), ())),
                                 preferred_element_type=jnp.float32)
                 for h in range(H_HEADS)]
        gt_ref[...] = lax.concatenate(grows + [zeros4], 0)
        cbv = jnp.sum(att_ref[...] * b1r_ref[...], axis=1, keepdims=True)
        cb8 = lax.concatenate([cbv, jnp.zeros((H_HEADS, 1), jnp.float32)], 0)
        cb_ref[...] = jnp.broadcast_to(cb8, (HP, 128))
        for h in range(H_HEADS):
            m_ref[h] = jnp.dot(w2r_ref[h], w1r_ref[h],
                               preferred_element_type=jnp.float32)

    @pl.when(p == 0)
    def _scatter():
        @pl.when(j == 0)
        def _init():
            ys_ref[...] = jnp.zeros_like(ys_ref)

        idxv = idxr_ref[0, 0]
        oht = (lax.broadcasted_iota(jnp.int32, (EP, BN), 0) == idxv
               ).astype(jnp.bfloat16)
        ohts_ref[j] = oht
        xb = x_ref[0].astype(jnp.bfloat16)
        xbf_ref[j] = xb
        vtb = vt_ref[...].astype(jnp.bfloat16)
        logits = jnp.dot(vtb, xb, preferred_element_type=jnp.float32)
        logits = logits + cb_ref[:, :1]
        ex = jnp.exp(logits)
        exb = ex.astype(jnp.bfloat16)
        ls = lax.concatenate(
            [xb * exb[h : h + 1, :] for h in range(H_HEADS)] + [exb], 0)
        ys_ref[...] += lax.dot_general(ls, oht, (((1,), (1,)), ((), ())),
                                       preferred_element_type=jnp.float32)

    @pl.when((p == 1) & (j == 0))
    def _edge():
        s = ys_ref[4 * C_IN : 4 * C_IN + HP]
        cnt = jnp.maximum(s[H_HEADS : H_HEADS + 1, :], 1.0)
        cinv = 1.0 / cnt
        dn = jnp.sum(s, axis=1, keepdims=True)
        sn = s * (cinv / dn)
        acc = lax.dot_general(gt_ref[...], sn, (((0,), (0,)), ((), ())),
                              preferred_element_type=jnp.float32)
        for h in range(H_HEADS):
            dnh = jnp.sum(s[h : h + 1, :], axis=1, keepdims=True)
            zh = ys_ref[h * C_IN : (h + 1) * C_IN] * (cinv / dnh)
            acc = acc + jnp.dot(m_ref[h], zh,
                                preferred_element_type=jnp.float32)
        et_ref[...] = acc + b2_ref[...]

    @pl.when(p == 1)
    def _output():
        oht = ohts_ref[j]
        etb = et_ref[...].astype(jnp.bfloat16)
        g = jnp.dot(etb, oht, preferred_element_type=jnp.float32)
        wrb = wr_ref[...].astype(jnp.bfloat16)
        r = jnp.dot(wrb, xbf_ref[j], preferred_element_type=jnp.float32)
        v = g + r + br_ref[...]
        o_ref[0] = jnp.where(v > 0, v, jnp.exp(v) - 1.0)


def kernel(x, hyperedge_labels, W1, b1, W2, b2, attention, Wr, br):
    B, C, H, W = x.shape
    N = H * W
    x3 = x.reshape(B, C, N)
    idx32 = hyperedge_labels.astype(jnp.int32)
    idxr = idx32.reshape(B, NB, 1, BN)
    idxc = idx32.reshape(B, NB, BN, 1)

    w1r = W1.reshape(H_HEADS, C_OUT, C)
    b1r = b1.reshape(H_HEADS, C_OUT)
    att = attention.reshape(H_HEADS, C_OUT)
    w2r = jnp.transpose(W2.reshape(C_OUT, H_HEADS, C_OUT), (1, 0, 2))
    b2c = b2.reshape(C_OUT, 1)
    brc = br.reshape(C_OUT, 1)

    out = pl.pallas_call(
        _fused_body,
        grid=(B, 2, NB),
        in_specs=[
            pl.BlockSpec((1, C, BN),
                         lambda b, p, j: (b, 0, j * (1 - p) + (NB - 1) * p)),
            pl.BlockSpec((1, 1, 1, BN), lambda b, p, j: (b, j, 0, 0)),
            pl.BlockSpec((H_HEADS, C_OUT, C_IN), lambda b, p, j: (0, 0, 0)),
            pl.BlockSpec((H_HEADS, C_OUT), lambda b, p, j: (0, 0)),
            pl.BlockSpec((H_HEADS, C_OUT), lambda b, p, j: (0, 0)),
            pl.BlockSpec((H_HEADS, C_OUT, C_IN), lambda b, p, j: (0, 0, 0)),
            pl.BlockSpec((C_OUT, 1), lambda b, p, j: (0, 0)),
            pl.BlockSpec((C_OUT, C_IN), lambda b, p, j: (0, 0)),
            pl.BlockSpec((C_OUT, 1), lambda b, p, j: (0, 0)),
        ],
        out_specs=pl.BlockSpec((1, C_OUT, BN), lambda b, p, j: (b, 0, j)),
        out_shape=jax.ShapeDtypeStruct((B, C_OUT, N), jnp.float32),
        scratch_shapes=[
            pltpu.VMEM((NB, C_IN, BN), jnp.bfloat16),
            pltpu.VMEM((NB, EP, BN), jnp.bfloat16),
            pltpu.VMEM((H_HEADS * C_IN + HP, EP), jnp.float32),
            pltpu.VMEM((C_OUT, EP), jnp.float32),
            pltpu.VMEM((HP, C_IN), jnp.float32),
            pltpu.VMEM((HP, 128), jnp.float32),
            pltpu.VMEM((H_HEADS, C_OUT, C_IN), jnp.float32),
            pltpu.VMEM((HP, C_IN), jnp.float32),
        ],
        compiler_params=pltpu.CompilerParams(
            vmem_limit_bytes=128 * 1024 * 1024,
        ),
    )(x3, idxr, w1r, b1r, att, w2r, b2c, Wr, brc)

    return out.reshape(B, C_OUT, H, W)


# speedup vs baseline: 1.3115x; 1.0278x over previous
import jax
import jax.numpy as jnp
from jax import lax
from jax.experimental import pallas as pl
from jax.experimental.pallas import tpu as pltpu

H_HEADS = 4
HP = 8
C_IN = 192
C_OUT = 192
E_EDGES = 200
EP = 208
BN = 7168
N_TOT = 224 * 224
NB = N_TOT // BN


def _fused_body(x_ref, idxr_ref, w1r_ref, b1r_ref, att_ref, w2r_ref,
                b2_ref, wr_ref, br_ref, o_ref, xbf_ref, ys_ref,
                et_ref, vt_ref, cb_ref, m_ref, gt_ref):
    b = pl.program_id(0)
    p = pl.program_id(1)
    j = pl.program_id(2)

    @pl.when((b == 0) & (p == 0) & (j == 0))
    def _prep():
        zeros4 = jnp.zeros((H_HEADS, C_IN), dtype=jnp.float32)
        vrows = [jnp.dot(att_ref[h : h + 1, :], w1r_ref[h],
                         preferred_element_type=jnp.float32)
                 for h in range(H_HEADS)]
        vt_ref[...] = lax.concatenate(vrows + [zeros4], 0)
        grows = [lax.dot_general(b1r_ref[h : h + 1, :], w2r_ref[h],
                                 (((1,), (1,)), ((), ())),
                                 preferred_element_type=jnp.float32)
                 for h in range(H_HEADS)]
        gt_ref[...] = lax.concatenate(grows + [zeros4], 0)
        cbv = jnp.sum(att_ref[...] * b1r_ref[...], axis=1, keepdims=True)
        cb8 = lax.concatenate([cbv, jnp.zeros((H_HEADS, 1), jnp.float32)], 0)
        cb_ref[...] = jnp.broadcast_to(cb8, (HP, 128))
        for h in range(H_HEADS):
            m_ref[h] = jnp.dot(w2r_ref[h], w1r_ref[h],
                               preferred_element_type=jnp.float32)

    @pl.when(p == 0)
    def _scatter():
        @pl.when(j == 0)
        def _init():
            ys_ref[...] = jnp.zeros_like(ys_ref)

        idxv = idxr_ref[0, 0]
        oht = (lax.broadcasted_iota(jnp.int32, (EP, BN), 0) == idxv
               ).astype(jnp.bfloat16)
        xb = x_ref[0].astype(jnp.bfloat16)
        xbf_ref[j] = xb
        vtb = vt_ref[...].astype(jnp.bfloat16)
        logits = jnp.dot(vtb, xb, preferred_element_type=jnp.float32)
        logits = logits + cb_ref[:, :1]
        ex = jnp.exp(logits)
        exb = ex.astype(jnp.bfloat16)
        ls = lax.concatenate(
            [xb * exb[h : h + 1, :] for h in range(H_HEADS)] + [exb], 0)
        ys_ref[...] += lax.dot_general(ls, oht, (((1,), (1,)), ((), ())),
                                       preferred_element_type=jnp.float32)

    @pl.when((p == 1) & (j == 0))
    def _edge():
        s = ys_ref[4 * C_IN : 4 * C_IN + HP]
        cnt = jnp.maximum(s[H_HEADS : H_HEADS + 1, :], 1.0)
        cinv = 1.0 / cnt
        dn = jnp.sum(s, axis=1, keepdims=True)
        sn = s * (cinv / dn)
        acc = lax.dot_general(gt_ref[...], sn, (((0,), (0,)), ((), ())),
                              preferred_element_type=jnp.float32)
        for h in range(H_HEADS):
            dnh = jnp.sum(s[h : h + 1, :], axis=1, keepdims=True)
            zh = ys_ref[h * C_IN : (h + 1) * C_IN] * (cinv / dnh)
            acc = acc + jnp.dot(m_ref[h], zh,
                                preferred_element_type=jnp.float32)
        et_ref[...] = acc + b2_ref[...]

    @pl.when(p == 1)
    def _output():
        idxv = idxr_ref[0, 0]
        oht = (lax.broadcasted_iota(jnp.int32, (EP, BN), 0) == idxv
               ).astype(jnp.bfloat16)
        etb = et_ref[...].astype(jnp.bfloat16)
        g = jnp.dot(etb, oht, preferred_element_type=jnp.float32)
        wrb = wr_ref[...].astype(jnp.bfloat16)
        r = jnp.dot(wrb, xbf_ref[j], preferred_element_type=jnp.float32)
        v = g + r + br_ref[...]
        o_ref[0] = jnp.where(v > 0, v, jnp.exp(v) - 1.0)


def kernel(x, hyperedge_labels, W1, b1, W2, b2, attention, Wr, br):
    B, C, H, W = x.shape
    N = H * W
    x3 = x.reshape(B, C, N)
    idx32 = hyperedge_labels.astype(jnp.int32)
    idxr = idx32.reshape(B, NB, 1, BN)
    idxc = idx32.reshape(B, NB, BN, 1)

    w1r = W1.reshape(H_HEADS, C_OUT, C)
    b1r = b1.reshape(H_HEADS, C_OUT)
    att = attention.reshape(H_HEADS, C_OUT)
    w2r = jnp.transpose(W2.reshape(C_OUT, H_HEADS, C_OUT), (1, 0, 2))
    b2c = b2.reshape(C_OUT, 1)
    brc = br.reshape(C_OUT, 1)

    out = pl.pallas_call(
        _fused_body,
        grid=(B, 2, NB),
        in_specs=[
            pl.BlockSpec((1, C, BN),
                         lambda b, p, j: (b, 0, j * (1 - p) + (NB - 1) * p)),
            pl.BlockSpec((1, 1, 1, BN), lambda b, p, j: (b, j, 0, 0)),
            pl.BlockSpec((H_HEADS, C_OUT, C_IN), lambda b, p, j: (0, 0, 0)),
            pl.BlockSpec((H_HEADS, C_OUT), lambda b, p, j: (0, 0)),
            pl.BlockSpec((H_HEADS, C_OUT), lambda b, p, j: (0, 0)),
            pl.BlockSpec((H_HEADS, C_OUT, C_IN), lambda b, p, j: (0, 0, 0)),
            pl.BlockSpec((C_OUT, 1), lambda b, p, j: (0, 0)),
            pl.BlockSpec((C_OUT, C_IN), lambda b, p, j: (0, 0)),
            pl.BlockSpec((C_OUT, 1), lambda b, p, j: (0, 0)),
        ],
        out_specs=pl.BlockSpec((1, C_OUT, BN), lambda b, p, j: (b, 0, j)),
        out_shape=jax.ShapeDtypeStruct((B, C_OUT, N), jnp.float32),
        scratch_shapes=[
            pltpu.VMEM((NB, C_IN, BN), jnp.bfloat16),
            pltpu.VMEM((H_HEADS * C_IN + HP, EP), jnp.float32),
            pltpu.VMEM((C_OUT, EP), jnp.float32),
            pltpu.VMEM((HP, C_IN), jnp.float32),
            pltpu.VMEM((HP, 128), jnp.float32),
            pltpu.VMEM((H_HEADS, C_OUT, C_IN), jnp.float32),
            pltpu.VMEM((HP, C_IN), jnp.float32),
        ],
        compiler_params=pltpu.CompilerParams(
            vmem_limit_bytes=128 * 1024 * 1024,
        ),
    )(x3, idxr, w1r, b1r, att, w2r, b2c, Wr, brc)

    return out.reshape(B, C_OUT, H, W)


# speedup vs baseline: 1.3226x; 1.0085x over previous
import jax
import jax.numpy as jnp
from jax import lax
from jax.experimental import pallas as pl
from jax.experimental.pallas import tpu as pltpu

H_HEADS = 4
HP = 8
C_IN = 192
C_OUT = 192
E_EDGES = 200
EP = 208
BN = 3584
N_TOT = 224 * 224
NB = N_TOT // BN
NBATCH = 2


def _fused_body(x_ref, idxa_ref, idxb_ref, w1r_ref, b1r_ref, att_ref,
                w2r_ref, b2_ref, wr_ref, br_ref, o_ref, xbf_ref, ys_ref,
                et_ref, vt_ref, cb_ref, m_ref, gt_ref):
    bb = pl.program_id(0)
    j = pl.program_id(1)

    @pl.when((bb == 0) & (j == 0))
    def _prep():
        zeros4 = jnp.zeros((H_HEADS, C_IN), dtype=jnp.float32)
        vrows = [jnp.dot(att_ref[h : h + 1, :], w1r_ref[h],
                         preferred_element_type=jnp.float32)
                 for h in range(H_HEADS)]
        vt_ref[...] = lax.concatenate(vrows + [zeros4], 0)
        grows = [lax.dot_general(b1r_ref[h : h + 1, :], w2r_ref[h],
                                 (((1,), (1,)), ((), ())),
                                 preferred_element_type=jnp.float32)
                 for h in range(H_HEADS)]
        gt_ref[...] = lax.concatenate(grows + [zeros4], 0)
        cbv = jnp.sum(att_ref[...] * b1r_ref[...], axis=1, keepdims=True)
        cb8 = lax.concatenate([cbv, jnp.zeros((H_HEADS, 1), jnp.float32)], 0)
        cb_ref[...] = jnp.broadcast_to(cb8, (HP, 128))
        for h in range(H_HEADS):
            m_ref[h] = jnp.dot(w2r_ref[h], w1r_ref[h],
                               preferred_element_type=jnp.float32)

    @pl.when((bb >= 1) & (j == 0))
    def _edge():
        s = ys_ref[4 * C_IN : 4 * C_IN + HP]
        cnt = jnp.maximum(s[H_HEADS : H_HEADS + 1, :], 1.0)
        cinv = 1.0 / cnt
        sn = s * (cinv / jnp.sum(s, axis=1, keepdims=True))
        acc = lax.dot_general(gt_ref[...], sn, (((0,), (0,)), ((), ())),
                              preferred_element_type=jnp.float32)
        for h in range(H_HEADS):
            dnh = jnp.sum(s[h : h + 1, :], axis=1, keepdims=True)
            zh = ys_ref[h * C_IN : (h + 1) * C_IN] * (cinv / dnh)
            acc = acc + jnp.dot(m_ref[h], zh,
                                preferred_element_type=jnp.float32)
        et_ref[...] = acc + b2_ref[...]

    @pl.when(bb < NBATCH)
    def _scatter():
        @pl.when(j == 0)
        def _init():
            ys_ref[...] = jnp.zeros_like(ys_ref)

        idxv = idxa_ref[0, 0]
        oht = (lax.broadcasted_iota(jnp.int32, (EP, BN), 0) == idxv
               ).astype(jnp.bfloat16)
        xb = x_ref[0].astype(jnp.bfloat16)
        xbf_ref[lax.rem(bb, 2), j] = xb
        vtb = vt_ref[...].astype(jnp.bfloat16)
        logits = jnp.dot(vtb, xb, preferred_element_type=jnp.float32)
        logits = logits + cb_ref[:, :1]
        ex = jnp.exp(logits)
        exb = ex.astype(jnp.bfloat16)
        ls = lax.concatenate(
            [xb * exb[h : h + 1, :] for h in range(H_HEADS)] + [exb], 0)
        ys_ref[...] += lax.dot_general(ls, oht, (((1,), (1,)), ((), ())),
                                       preferred_element_type=jnp.float32)

    @pl.when(bb >= 1)
    def _output():
        idxv = idxb_ref[0, 0]
        oht = (lax.broadcasted_iota(jnp.int32, (EP, BN), 0) == idxv
               ).astype(jnp.bfloat16)
        etb = et_ref[...].astype(jnp.bfloat16)
        g = jnp.dot(etb, oht, preferred_element_type=jnp.float32)
        wrb = wr_ref[...].astype(jnp.bfloat16)
        r = jnp.dot(wrb, xbf_ref[lax.rem(bb + 1, 2), j],
                    preferred_element_type=jnp.float32)
        v = g + r + br_ref[...]
        o_ref[0] = jnp.where(v > 0, v, jnp.exp(v) - 1.0)


def kernel(x, hyperedge_labels, W1, b1, W2, b2, attention, Wr, br):
    B, C, H, W = x.shape
    N = H * W
    x3 = x.reshape(B, C, N)
    idxr = hyperedge_labels.astype(jnp.int32).reshape(B, NB, 1, BN)

    w1r = W1.reshape(H_HEADS, C_OUT, C)
    b1r = b1.reshape(H_HEADS, C_OUT)
    att = attention.reshape(H_HEADS, C_OUT)
    w2r = jnp.transpose(W2.reshape(C_OUT, H_HEADS, C_OUT), (1, 0, 2))
    b2c = b2.reshape(C_OUT, 1)
    brc = br.reshape(C_OUT, 1)

    out = pl.pallas_call(
        _fused_body,
        grid=(B + 1, NB),
        in_specs=[
            pl.BlockSpec(
                (1, C, BN),
                lambda bb, j: (bb - bb // NBATCH, 0,
                               j * (1 - bb // NBATCH)
                               + (NB - 1) * (bb // NBATCH))),
            pl.BlockSpec(
                (1, 1, 1, BN),
                lambda bb, j: (bb - bb // NBATCH, j, 0, 0)),
            pl.BlockSpec(
                (1, 1, 1, BN),
                lambda bb, j: (bb - 1 + (1 - jnp.minimum(bb, 1)), j, 0, 0)),
            pl.BlockSpec((H_HEADS, C_OUT, C_IN), lambda bb, j: (0, 0, 0)),
            pl.BlockSpec((H_HEADS, C_OUT), lambda bb, j: (0, 0)),
            pl.BlockSpec((H_HEADS, C_OUT), lambda bb, j: (0, 0)),
            pl.BlockSpec((H_HEADS, C_OUT, C_IN), lambda bb, j: (0, 0, 0)),
            pl.BlockSpec((C_OUT, 1), lambda bb, j: (0, 0)),
            pl.BlockSpec((C_OUT, C_IN), lambda bb, j: (0, 0)),
            pl.BlockSpec((C_OUT, 1), lambda bb, j: (0, 0)),
        ],
        out_specs=pl.BlockSpec(
            (1, C_OUT, BN),
            lambda bb, j: ((bb - 1) * jnp.minimum(bb, 1), 0,
                           j * jnp.minimum(bb, 1))),
        out_shape=jax.ShapeDtypeStruct((B, C_OUT, N), jnp.float32),
        scratch_shapes=[
            pltpu.VMEM((2, NB, C_IN, BN), jnp.bfloat16),
            pltpu.VMEM((H_HEADS * C_IN + HP, EP), jnp.float32),
            pltpu.VMEM((C_OUT, EP), jnp.float32),
            pltpu.VMEM((HP, C_IN), jnp.float32),
            pltpu.VMEM((HP, 128), jnp.float32),
            pltpu.VMEM((H_HEADS, C_OUT, C_IN), jnp.float32),
            pltpu.VMEM((HP, C_IN), jnp.float32),
        ],
        compiler_params=pltpu.CompilerParams(
            vmem_limit_bytes=128 * 1024 * 1024,
        ),
    )(x3, idxr, idxr, w1r, b1r, att, w2r, b2c, Wr, brc)

    return out.reshape(B, C_OUT, H, W)
